# Initial kernel scaffold; baseline (speedup 1.0000x reference)
import functools

import jax
import jax.numpy as jnp
from jax import lax
from jax.experimental import pallas as pl
from jax.experimental.pallas import tpu as pltpu
from jax.experimental.pallas import tpu_sc as plsc

N = 10000
E = 320000
H = 128
G = 64

NC = 2
NS = 16
NW = NC * NS
EPW = E // NW
K = 80
NCH = EPW // K
RPT = N // NS
ZR = 125

_mesh = plsc.VectorSubcoreMesh(core_axis_name="c", subcore_axis_name="s")


@functools.partial(
    pl.kernel,
    mesh=_mesh,
    out_type=jax.ShapeDtypeStruct((NC, N, 16), jnp.float32),
    scratch_types=[
        pltpu.VMEM((K,), jnp.int32),
        pltpu.VMEM((K, 16), jnp.float32),
        pltpu.VMEM((RPT, 16), jnp.float32),
        pltpu.VMEM_SHARED((N, 16), jnp.float32),
    ],
)
def _sc_deg(dst, out, dst_v, ones_v, stage_v, acc):
    cid = lax.axis_index("c")
    sid = lax.axis_index("s")
    wid = sid * NC + cid

    e0 = jnp.where(lax.iota(jnp.int32, 16) == 0, 1.0, 0.0).astype(jnp.float32)
    zv = jnp.zeros((16,), jnp.float32)

    def fill(i, carry):
        ones_v[i] = e0
        return carry
    lax.fori_loop(0, K, fill, 0)

    def zfill(i, carry):
        stage_v[i] = zv
        return carry
    lax.fori_loop(0, RPT, zfill, 0)

    pltpu.sync_copy(stage_v, acc.at[pl.ds(sid * RPT, RPT)])
    plsc.subcore_barrier()

    ebase = wid * EPW

    def body(c, carry):
        pltpu.sync_copy(dst.at[pl.ds(ebase + c * K, K)], dst_v)
        pltpu.sync_copy(ones_v, acc.at[dst_v], add=True)
        return carry
    lax.fori_loop(0, NCH, body, 0)

    plsc.subcore_barrier()
    pltpu.sync_copy(acc.at[pl.ds(sid * RPT, RPT)], stage_v)
    pltpu.sync_copy(stage_v, out.at[cid, pl.ds(sid * RPT, RPT)])


@functools.partial(
    pl.kernel,
    mesh=_mesh,
    out_type=jax.ShapeDtypeStruct((NC, N, H), jnp.float32),
    scratch_types=[
        pltpu.VMEM((K,), jnp.int32),
        pltpu.VMEM((K,), jnp.int32),
        pltpu.VMEM((K, H), jnp.float32),
        pltpu.VMEM((ZR, H), jnp.float32),
        pltpu.VMEM_SHARED((N, H), jnp.float32),
        pltpu.SemaphoreType.DMA,
    ],
)
def _sc_agg(hws, src, dst, out, src_v, dst_v, rows_v, stage_v, acc, sem):
    cid = lax.axis_index("c")
    sid = lax.axis_index("s")
    wid = sid * NC + cid

    zv = jnp.zeros((16,), jnp.float32)

    def zfill(i, carry):
        for j in range(H // 16):
            stage_v[i, pl.ds(j * 16, 16)] = zv
        return carry
    lax.fori_loop(0, ZR, zfill, 0)

    def zstripe(t, carry):
        pltpu.sync_copy(stage_v, acc.at[pl.ds(sid * RPT + t * ZR, ZR)])
        return carry
    lax.fori_loop(0, RPT // ZR, zstripe, 0)
    plsc.subcore_barrier()

    ebase = wid * EPW

    def body(c, carry):
        b = ebase + c * K
        pltpu.sync_copy(src.at[pl.ds(b, K)], src_v)
        pltpu.sync_copy(dst.at[pl.ds(b, K)], dst_v)
        pltpu.async_copy(hws.at[src_v], rows_v, sem).wait()
        pltpu.sync_copy(rows_v, acc.at[dst_v], add=True)
        return carry
    lax.fori_loop(0, NCH, body, 0)

    plsc.subcore_barrier()

    def wb(t, carry):
        r0 = sid * RPT + t * ZR
        pltpu.sync_copy(acc.at[pl.ds(r0, ZR)], stage_v)
        pltpu.sync_copy(stage_v, out.at[cid, pl.ds(r0, ZR)])
        return carry
    lax.fori_loop(0, RPT // ZR, wb, 0)


R = 1000


def _dot(a, b):
    return jnp.dot(a, b, preferred_element_type=jnp.float32)


def _pre_body(x_ref, w0_ref, b0_ref, w1_ref, b1_ref, gw1_ref, dp_ref,
              hws_ref, dinv_ref):
    deg = dp_ref[0, :, 0] + dp_ref[1, :, 0] + 1.0
    dinv = lax.rsqrt(deg)[:, None]
    t = jnp.maximum(_dot(x_ref[...], w0_ref[...]) + b0_ref[...], 0.0)
    h0 = _dot(t, w1_ref[...]) + b1_ref[...]
    hws_ref[...] = _dot(h0, gw1_ref[...]) * dinv
    dinv_ref[...] = dinv


def _tc_pre(x, W0, b0, W1, b1, gW1, degp):
    return pl.pallas_call(
        _pre_body,
        grid=(N // R,),
        in_specs=[
            pl.BlockSpec((R, H), lambda i: (i, 0)),
            pl.BlockSpec((H, H), lambda i: (0, 0)),
            pl.BlockSpec((1, H), lambda i: (0, 0)),
            pl.BlockSpec((H, H), lambda i: (0, 0)),
            pl.BlockSpec((1, H), lambda i: (0, 0)),
            pl.BlockSpec((H, H), lambda i: (0, 0)),
            pl.BlockSpec((NC, R, 16), lambda i: (0, i, 0)),
        ],
        out_specs=[
            pl.BlockSpec((R, H), lambda i: (i, 0)),
            pl.BlockSpec((R, 1), lambda i: (i, 0)),
        ],
        out_shape=[
            jax.ShapeDtypeStruct((N, H), jnp.float32),
            jax.ShapeDtypeStruct((N, 1), jnp.float32),
        ],
    )(x, W0, b0.reshape(1, H), W1, b1.reshape(1, H), gW1, degp)


def _layer_body(p_ref, hws_ref, dinv_ref, b_ref, w_ref, out_ref):
    dinv = dinv_ref[...]
    y = jnp.maximum(
        (p_ref[0] + p_ref[1] + hws_ref[...]) * dinv + b_ref[...], 0.0)
    out_ref[...] = _dot(y, w_ref[...]) * dinv


def _tc_layer(parts, hws, dinv, b, Wn):
    return pl.pallas_call(
        _layer_body,
        grid=(N // R,),
        in_specs=[
            pl.BlockSpec((NC, R, H), lambda i: (0, i, 0)),
            pl.BlockSpec((R, H), lambda i: (i, 0)),
            pl.BlockSpec((R, 1), lambda i: (i, 0)),
            pl.BlockSpec((1, H), lambda i: (0, 0)),
            pl.BlockSpec((H, H), lambda i: (0, 0)),
        ],
        out_specs=pl.BlockSpec((R, H), lambda i: (i, 0)),
        out_shape=jax.ShapeDtypeStruct((N, H), jnp.float32),
    )(parts, hws, dinv, b.reshape(1, H), Wn)


def _final_body(p_ref, hws_ref, dinv_ref, b_ref, w1_ref, b1_ref, w2_ref,
                b2_ref, bat_ref, out_ref):
    y = jnp.maximum(
        (p_ref[0] + p_ref[1] + hws_ref[...]) * dinv_ref[...] + b_ref[...],
        0.0)
    t = jnp.maximum(_dot(y, w1_ref[...]) + b1_ref[...], 0.0)
    z = _dot(t, w2_ref[...]) + b2_ref[...]
    onehot = (bat_ref[...] == lax.broadcasted_iota(jnp.int32, (R, G), 1))
    contrib = lax.dot_general(onehot.astype(jnp.float32), z,
                              (((0,), (0,)), ((), ())),
                              preferred_element_type=jnp.float32)

    @pl.when(pl.program_id(0) == 0)
    def _():
        out_ref[...] = jnp.zeros_like(out_ref)

    out_ref[...] += contrib


def _tc_final(parts, hws, dinv, b, l1W, l1b, l2W, l2b, batch2d):
    return pl.pallas_call(
        _final_body,
        grid=(N // R,),
        in_specs=[
            pl.BlockSpec((NC, R, H), lambda i: (0, i, 0)),
            pl.BlockSpec((R, H), lambda i: (i, 0)),
            pl.BlockSpec((R, 1), lambda i: (i, 0)),
            pl.BlockSpec((1, H), lambda i: (0, 0)),
            pl.BlockSpec((H, H // 2), lambda i: (0, 0)),
            pl.BlockSpec((1, H // 2), lambda i: (0, 0)),
            pl.BlockSpec((H // 2, 1), lambda i: (0, 0)),
            pl.BlockSpec((1, 1), lambda i: (0, 0)),
            pl.BlockSpec((R, 1), lambda i: (i, 0)),
        ],
        out_specs=pl.BlockSpec((G, 1), lambda i: (0, 0)),
        out_shape=jax.ShapeDtypeStruct((G, 1), jnp.float32),
    )(parts, hws, dinv, b.reshape(1, H), l1W, l1b.reshape(1, H // 2), l2W,
      l2b.reshape(1, 1), batch2d)


def kernel(x, pos, edge_index, batch, W0, b0, W1, b1, gW1, gb1, gW2, gb2,
           gW3, gb3, l1W, l1b, l2W, l2b):
    src = edge_index[0]
    dst = edge_index[1]

    degp = _sc_deg(dst)
    hws1, dinv = _tc_pre(x, W0, b0, W1, b1, gW1, degp)
    p1 = _sc_agg(hws1, src, dst)
    hws2 = _tc_layer(p1, hws1, dinv, gb1, gW2)
    p2 = _sc_agg(hws2, src, dst)
    hws3 = _tc_layer(p2, hws2, dinv, gb2, gW3)
    p3 = _sc_agg(hws3, src, dst)
    return _tc_final(p3, hws3, dinv, gb3, l1W, l1b, l2W, l2b,
                     batch.reshape(N, 1))


# speedup vs baseline: 10.7454x; 10.7454x over previous
import functools

import jax
import jax.numpy as jnp
from jax import lax
from jax.experimental import pallas as pl
from jax.experimental.pallas import tpu as pltpu
from jax.experimental.pallas import tpu_sc as plsc

N = 10000
E = 320000
H = 128
G = 64

NC = 2
NS = 16
NW = NC * NS
EPW = E // NW
K = 80
NCH = EPW // K
NP = 10240
RPT = NP // NS
ZR = 128

_mesh = plsc.VectorSubcoreMesh(core_axis_name="c", subcore_axis_name="s")


@functools.partial(
    pl.kernel,
    mesh=_mesh,
    out_type=jax.ShapeDtypeStruct((NC, NP, H), jnp.float32),
    scratch_types=[
        pltpu.VMEM((K,), jnp.int32),
        pltpu.VMEM((K, H), jnp.float32),
        pltpu.VMEM((ZR, H), jnp.float32),
        pltpu.VMEM_SHARED((NP, H), jnp.float32),
    ],
)
def _sc_deg(ones, zblk, dst, out, dst_v, ones_v, stage_v, acc):
    cid = lax.axis_index("c")
    sid = lax.axis_index("s")
    wid = sid * NC + cid

    pltpu.sync_copy(ones, ones_v)
    pltpu.sync_copy(zblk, stage_v)

    def zstripe(t, carry):
        pltpu.sync_copy(stage_v, acc.at[pl.ds(sid * RPT + t * ZR, ZR)])
        return carry
    lax.fori_loop(0, RPT // ZR, zstripe, 0)
    plsc.subcore_barrier()

    ebase = wid * EPW

    def body(c, carry):
        pltpu.sync_copy(dst.at[pl.ds(ebase + c * K, K)], dst_v)
        pltpu.sync_copy(ones_v, acc.at[dst_v], add=True)
        return carry
    lax.fori_loop(0, NCH, body, 0)

    plsc.subcore_barrier()

    def wb(t, carry):
        r0 = sid * RPT + t * ZR
        pltpu.sync_copy(acc.at[pl.ds(r0, ZR)], stage_v)
        pltpu.sync_copy(stage_v, out.at[cid, pl.ds(r0, ZR)])
        return carry
    lax.fori_loop(0, RPT // ZR, wb, 0)


@functools.partial(
    pl.kernel,
    mesh=_mesh,
    out_type=jax.ShapeDtypeStruct((NC, NP, H), jnp.float32),
    scratch_types=[
        pltpu.VMEM((K,), jnp.int32),
        pltpu.VMEM((K,), jnp.int32),
        pltpu.VMEM((K, H), jnp.float32),
        pltpu.VMEM((ZR, H), jnp.float32),
        pltpu.VMEM_SHARED((NP, H), jnp.float32),
        pltpu.SemaphoreType.DMA,
    ],
)
def _sc_agg(zblk, hws, src, dst, out, src_v, dst_v, rows_v, stage_v, acc,
            sem):
    cid = lax.axis_index("c")
    sid = lax.axis_index("s")
    wid = sid * NC + cid

    pltpu.sync_copy(zblk, stage_v)

    def zstripe(t, carry):
        pltpu.sync_copy(stage_v, acc.at[pl.ds(sid * RPT + t * ZR, ZR)])
        return carry
    lax.fori_loop(0, RPT // ZR, zstripe, 0)
    plsc.subcore_barrier()

    ebase = wid * EPW

    def body(c, carry):
        b = ebase + c * K
        pltpu.sync_copy(src.at[pl.ds(b, K)], src_v)
        pltpu.sync_copy(dst.at[pl.ds(b, K)], dst_v)
        pltpu.async_copy(hws.at[src_v], rows_v, sem).wait()
        pltpu.sync_copy(rows_v, acc.at[dst_v], add=True)
        return carry
    lax.fori_loop(0, NCH, body, 0)

    plsc.subcore_barrier()

    def wb(t, carry):
        r0 = sid * RPT + t * ZR
        pltpu.sync_copy(acc.at[pl.ds(r0, ZR)], stage_v)
        pltpu.sync_copy(stage_v, out.at[cid, pl.ds(r0, ZR)])
        return carry
    lax.fori_loop(0, RPT // ZR, wb, 0)


R = 1000


def _dot(a, b):
    return jnp.dot(a, b, preferred_element_type=jnp.float32)


def _pre_body(x_ref, w0_ref, b0_ref, w1_ref, b1_ref, gw1_ref, dp_ref,
              hws_ref, dinv_ref):
    deg = dp_ref[0, :, 0] + dp_ref[1, :, 0] + 1.0
    dinv = lax.rsqrt(deg)[:, None]
    t = jnp.maximum(_dot(x_ref[...], w0_ref[...]) + b0_ref[...], 0.0)
    h0 = _dot(t, w1_ref[...]) + b1_ref[...]
    hws_ref[...] = _dot(h0, gw1_ref[...]) * dinv
    dinv_ref[...] = dinv


def _tc_pre(x, W0, b0, W1, b1, gW1, degp):
    return pl.pallas_call(
        _pre_body,
        grid=(N // R,),
        in_specs=[
            pl.BlockSpec((R, H), lambda i: (i, 0)),
            pl.BlockSpec((H, H), lambda i: (0, 0)),
            pl.BlockSpec((1, H), lambda i: (0, 0)),
            pl.BlockSpec((H, H), lambda i: (0, 0)),
            pl.BlockSpec((1, H), lambda i: (0, 0)),
            pl.BlockSpec((H, H), lambda i: (0, 0)),
            pl.BlockSpec((NC, R, H), lambda i: (0, i, 0)),
        ],
        out_specs=[
            pl.BlockSpec((R, H), lambda i: (i, 0)),
            pl.BlockSpec((R, 1), lambda i: (i, 0)),
        ],
        out_shape=[
            jax.ShapeDtypeStruct((N, H), jnp.float32),
            jax.ShapeDtypeStruct((N, 1), jnp.float32),
        ],
    )(x, W0, b0.reshape(1, H), W1, b1.reshape(1, H), gW1, degp)


def _layer_body(p_ref, hws_ref, dinv_ref, b_ref, w_ref, out_ref):
    dinv = dinv_ref[...]
    y = jnp.maximum(
        (p_ref[0] + p_ref[1] + hws_ref[...]) * dinv + b_ref[...], 0.0)
    out_ref[...] = _dot(y, w_ref[...]) * dinv


def _tc_layer(parts, hws, dinv, b, Wn):
    return pl.pallas_call(
        _layer_body,
        grid=(N // R,),
        in_specs=[
            pl.BlockSpec((NC, R, H), lambda i: (0, i, 0)),
            pl.BlockSpec((R, H), lambda i: (i, 0)),
            pl.BlockSpec((R, 1), lambda i: (i, 0)),
            pl.BlockSpec((1, H), lambda i: (0, 0)),
            pl.BlockSpec((H, H), lambda i: (0, 0)),
        ],
        out_specs=pl.BlockSpec((R, H), lambda i: (i, 0)),
        out_shape=jax.ShapeDtypeStruct((N, H), jnp.float32),
    )(parts, hws, dinv, b.reshape(1, H), Wn)


def _final_body(p_ref, hws_ref, dinv_ref, b_ref, w1_ref, b1_ref, w2_ref,
                b2_ref, bat_ref, out_ref):
    y = jnp.maximum(
        (p_ref[0] + p_ref[1] + hws_ref[...]) * dinv_ref[...] + b_ref[...],
        0.0)
    t = jnp.maximum(_dot(y, w1_ref[...]) + b1_ref[...], 0.0)
    z = _dot(t, w2_ref[...]) + b2_ref[...]
    onehot = (bat_ref[...] == lax.broadcasted_iota(jnp.int32, (R, G), 1))
    contrib = lax.dot_general(onehot.astype(jnp.float32), z,
                              (((0,), (0,)), ((), ())),
                              preferred_element_type=jnp.float32)

    @pl.when(pl.program_id(0) == 0)
    def _():
        out_ref[...] = jnp.zeros_like(out_ref)

    out_ref[...] += contrib


def _tc_final(parts, hws, dinv, b, l1W, l1b, l2W, l2b, batch2d):
    return pl.pallas_call(
        _final_body,
        grid=(N // R,),
        in_specs=[
            pl.BlockSpec((NC, R, H), lambda i: (0, i, 0)),
            pl.BlockSpec((R, H), lambda i: (i, 0)),
            pl.BlockSpec((R, 1), lambda i: (i, 0)),
            pl.BlockSpec((1, H), lambda i: (0, 0)),
            pl.BlockSpec((H, H // 2), lambda i: (0, 0)),
            pl.BlockSpec((1, H // 2), lambda i: (0, 0)),
            pl.BlockSpec((H // 2, 1), lambda i: (0, 0)),
            pl.BlockSpec((1, 1), lambda i: (0, 0)),
            pl.BlockSpec((R, 1), lambda i: (i, 0)),
        ],
        out_specs=pl.BlockSpec((G, 1), lambda i: (0, 0)),
        out_shape=jax.ShapeDtypeStruct((G, 1), jnp.float32),
    )(parts, hws, dinv, b.reshape(1, H), l1W, l1b.reshape(1, H // 2), l2W,
      l2b.reshape(1, 1), batch2d)


def kernel(x, pos, edge_index, batch, W0, b0, W1, b1, gW1, gb1, gW2, gb2,
           gW3, gb3, l1W, l1b, l2W, l2b):
    src = edge_index[0]
    dst = edge_index[1]
    ones = jnp.ones((K, H), jnp.float32)
    zblk = jnp.zeros((ZR, H), jnp.float32)

    degp = _sc_deg(ones, zblk, dst)
    hws1, dinv = _tc_pre(x, W0, b0, W1, b1, gW1, degp)
    p1 = _sc_agg(zblk, hws1, src, dst)
    hws2 = _tc_layer(p1, hws1, dinv, gb1, gW2)
    p2 = _sc_agg(zblk, hws2, src, dst)
    hws3 = _tc_layer(p2, hws2, dinv, gb2, gW3)
    p3 = _sc_agg(zblk, hws3, src, dst)
    return _tc_final(p3, hws3, dinv, gb3, l1W, l1b, l2W, l2b,
                     batch.reshape(N, 1))


# speedup vs baseline: 20.3521x; 1.8940x over previous
import functools

import jax
import jax.numpy as jnp
from jax import lax
from jax.experimental import pallas as pl
from jax.experimental.pallas import tpu as pltpu
from jax.experimental.pallas import tpu_sc as plsc

N = 10000
E = 320000
H = 128
G = 64

NC = 2
NS = 16
NW = NC * NS
EPW = E // NW
K = 80
NCH = EPW // K
NP = 10240
RPT = NP // NS
ZR = 128

_mesh = plsc.VectorSubcoreMesh(core_axis_name="c", subcore_axis_name="s")


@functools.partial(
    pl.kernel,
    mesh=_mesh,
    out_type=jax.ShapeDtypeStruct((NC, NP, H), jnp.float32),
    scratch_types=[
        pltpu.VMEM((K,), jnp.int32),
        pltpu.VMEM((K, H), jnp.float32),
        pltpu.VMEM((ZR, H), jnp.float32),
        pltpu.VMEM_SHARED((NP, H), jnp.float32),
    ],
)
def _sc_deg(ones, zblk, dst, out, dst_v, ones_v, stage_v, acc):
    cid = lax.axis_index("c")
    sid = lax.axis_index("s")
    wid = sid * NC + cid

    pltpu.sync_copy(ones, ones_v)
    pltpu.sync_copy(zblk, stage_v)

    def zstripe(t, carry):
        pltpu.sync_copy(stage_v, acc.at[pl.ds(sid * RPT + t * ZR, ZR)])
        return carry
    lax.fori_loop(0, RPT // ZR, zstripe, 0)
    plsc.subcore_barrier()

    ebase = wid * EPW

    def body(c, carry):
        pltpu.sync_copy(dst.at[pl.ds(ebase + c * K, K)], dst_v)
        pltpu.sync_copy(ones_v, acc.at[dst_v], add=True)
        return carry
    lax.fori_loop(0, NCH, body, 0)

    plsc.subcore_barrier()

    def wb(t, carry):
        r0 = sid * RPT + t * ZR
        pltpu.sync_copy(acc.at[pl.ds(r0, ZR)], stage_v)
        pltpu.sync_copy(stage_v, out.at[cid, pl.ds(r0, ZR)])
        return carry
    lax.fori_loop(0, RPT // ZR, wb, 0)


@functools.partial(
    pl.kernel,
    mesh=_mesh,
    out_type=jax.ShapeDtypeStruct((NC, NP, H), jnp.float32),
    scratch_types=[
        pltpu.VMEM((EPW,), jnp.int32),
        pltpu.VMEM((K,), jnp.int32),
        pltpu.VMEM((K,), jnp.int32),
        pltpu.VMEM((K, H), jnp.float32),
        pltpu.VMEM((K, H), jnp.float32),
        pltpu.VMEM((ZR, H), jnp.float32),
        pltpu.VMEM_SHARED((NP, H), jnp.float32),
        pltpu.SemaphoreType.DMA,
        pltpu.SemaphoreType.DMA,
    ],
)
def _sc_agg(zblk, hws, src, dst, out, src_all, dst_a, dst_b, rows_a, rows_b,
            stage_v, acc, sem_a, sem_b):
    cid = lax.axis_index("c")
    sid = lax.axis_index("s")
    wid = sid * NC + cid

    pltpu.sync_copy(zblk, stage_v)

    def zstripe(t, carry):
        pltpu.sync_copy(stage_v, acc.at[pl.ds(sid * RPT + t * ZR, ZR)])
        return carry
    lax.fori_loop(0, RPT // ZR, zstripe, 0)

    ebase = wid * EPW
    pltpu.sync_copy(src.at[pl.ds(ebase, EPW)], src_all)
    plsc.subcore_barrier()

    def fire(c, rows, sem):
        return pltpu.async_copy(
            hws.at[src_all.at[pl.ds(c * K, K)]], rows, sem)

    def drain(c, rows, sem):
        pltpu.make_async_copy(
            hws.at[src_all.at[pl.ds(c * K, K)]], rows, sem).wait()

    fire(0, rows_a, sem_a)

    def body(i, carry):
        c0 = 2 * i
        fire(c0 + 1, rows_b, sem_b)
        pltpu.sync_copy(dst.at[pl.ds(ebase + c0 * K, K)], dst_a)
        drain(c0, rows_a, sem_a)
        pltpu.sync_copy(rows_a, acc.at[dst_a], add=True)
        fire(c0 + 2, rows_a, sem_a)
        pltpu.sync_copy(dst.at[pl.ds(ebase + (c0 + 1) * K, K)], dst_b)
        drain(c0 + 1, rows_b, sem_b)
        pltpu.sync_copy(rows_b, acc.at[dst_b], add=True)
        return carry
    lax.fori_loop(0, (NCH - 1) // 2, body, 0)

    cl = NCH - 1
    pltpu.sync_copy(dst.at[pl.ds(ebase + cl * K, K)], dst_a)
    drain(cl, rows_a, sem_a)
    pltpu.sync_copy(rows_a, acc.at[dst_a], add=True)

    plsc.subcore_barrier()

    def wb(t, carry):
        r0 = sid * RPT + t * ZR
        pltpu.sync_copy(acc.at[pl.ds(r0, ZR)], stage_v)
        pltpu.sync_copy(stage_v, out.at[cid, pl.ds(r0, ZR)])
        return carry
    lax.fori_loop(0, RPT // ZR, wb, 0)


R = 1000


def _dot(a, b):
    return jnp.dot(a, b, preferred_element_type=jnp.float32)


def _pre_body(x_ref, w0_ref, b0_ref, w1_ref, b1_ref, gw1_ref, dp_ref,
              hws_ref, dinv_ref):
    deg = dp_ref[0, :, 0] + dp_ref[1, :, 0] + 1.0
    dinv = lax.rsqrt(deg)[:, None]
    t = jnp.maximum(_dot(x_ref[...], w0_ref[...]) + b0_ref[...], 0.0)
    h0 = _dot(t, w1_ref[...]) + b1_ref[...]
    hws_ref[...] = _dot(h0, gw1_ref[...]) * dinv
    dinv_ref[...] = dinv


def _tc_pre(x, W0, b0, W1, b1, gW1, degp):
    return pl.pallas_call(
        _pre_body,
        grid=(N // R,),
        in_specs=[
            pl.BlockSpec((R, H), lambda i: (i, 0)),
            pl.BlockSpec((H, H), lambda i: (0, 0)),
            pl.BlockSpec((1, H), lambda i: (0, 0)),
            pl.BlockSpec((H, H), lambda i: (0, 0)),
            pl.BlockSpec((1, H), lambda i: (0, 0)),
            pl.BlockSpec((H, H), lambda i: (0, 0)),
            pl.BlockSpec((NC, R, H), lambda i: (0, i, 0)),
        ],
        out_specs=[
            pl.BlockSpec((R, H), lambda i: (i, 0)),
            pl.BlockSpec((R, 1), lambda i: (i, 0)),
        ],
        out_shape=[
            jax.ShapeDtypeStruct((N, H), jnp.float32),
            jax.ShapeDtypeStruct((N, 1), jnp.float32),
        ],
    )(x, W0, b0.reshape(1, H), W1, b1.reshape(1, H), gW1, degp)


def _layer_body(p_ref, hws_ref, dinv_ref, b_ref, w_ref, out_ref):
    dinv = dinv_ref[...]
    y = jnp.maximum(
        (p_ref[0] + p_ref[1] + hws_ref[...]) * dinv + b_ref[...], 0.0)
    out_ref[...] = _dot(y, w_ref[...]) * dinv


def _tc_layer(parts, hws, dinv, b, Wn):
    return pl.pallas_call(
        _layer_body,
        grid=(N // R,),
        in_specs=[
            pl.BlockSpec((NC, R, H), lambda i: (0, i, 0)),
            pl.BlockSpec((R, H), lambda i: (i, 0)),
            pl.BlockSpec((R, 1), lambda i: (i, 0)),
            pl.BlockSpec((1, H), lambda i: (0, 0)),
            pl.BlockSpec((H, H), lambda i: (0, 0)),
        ],
        out_specs=pl.BlockSpec((R, H), lambda i: (i, 0)),
        out_shape=jax.ShapeDtypeStruct((N, H), jnp.float32),
    )(parts, hws, dinv, b.reshape(1, H), Wn)


def _final_body(p_ref, hws_ref, dinv_ref, b_ref, w1_ref, b1_ref, w2_ref,
                b2_ref, bat_ref, out_ref):
    y = jnp.maximum(
        (p_ref[0] + p_ref[1] + hws_ref[...]) * dinv_ref[...] + b_ref[...],
        0.0)
    t = jnp.maximum(_dot(y, w1_ref[...]) + b1_ref[...], 0.0)
    z = _dot(t, w2_ref[...]) + b2_ref[...]
    onehot = (bat_ref[...] == lax.broadcasted_iota(jnp.int32, (R, G), 1))
    contrib = lax.dot_general(onehot.astype(jnp.float32), z,
                              (((0,), (0,)), ((), ())),
                              preferred_element_type=jnp.float32)

    @pl.when(pl.program_id(0) == 0)
    def _():
        out_ref[...] = jnp.zeros_like(out_ref)

    out_ref[...] += contrib


def _tc_final(parts, hws, dinv, b, l1W, l1b, l2W, l2b, batch2d):
    return pl.pallas_call(
        _final_body,
        grid=(N // R,),
        in_specs=[
            pl.BlockSpec((NC, R, H), lambda i: (0, i, 0)),
            pl.BlockSpec((R, H), lambda i: (i, 0)),
            pl.BlockSpec((R, 1), lambda i: (i, 0)),
            pl.BlockSpec((1, H), lambda i: (0, 0)),
            pl.BlockSpec((H, H // 2), lambda i: (0, 0)),
            pl.BlockSpec((1, H // 2), lambda i: (0, 0)),
            pl.BlockSpec((H // 2, 1), lambda i: (0, 0)),
            pl.BlockSpec((1, 1), lambda i: (0, 0)),
            pl.BlockSpec((R, 1), lambda i: (i, 0)),
        ],
        out_specs=pl.BlockSpec((G, 1), lambda i: (0, 0)),
        out_shape=jax.ShapeDtypeStruct((G, 1), jnp.float32),
    )(parts, hws, dinv, b.reshape(1, H), l1W, l1b.reshape(1, H // 2), l2W,
      l2b.reshape(1, 1), batch2d)


def kernel(x, pos, edge_index, batch, W0, b0, W1, b1, gW1, gb1, gW2, gb2,
           gW3, gb3, l1W, l1b, l2W, l2b):
    src = edge_index[0]
    dst = edge_index[1]
    ones = jnp.ones((K, H), jnp.float32)
    zblk = jnp.zeros((ZR, H), jnp.float32)

    degp = _sc_deg(ones, zblk, dst)
    hws1, dinv = _tc_pre(x, W0, b0, W1, b1, gW1, degp)
    p1 = _sc_agg(zblk, hws1, src, dst)
    hws2 = _tc_layer(p1, hws1, dinv, gb1, gW2)
    p2 = _sc_agg(zblk, hws2, src, dst)
    hws3 = _tc_layer(p2, hws2, dinv, gb2, gW3)
    p3 = _sc_agg(zblk, hws3, src, dst)
    return _tc_final(p3, hws3, dinv, gb3, l1W, l1b, l2W, l2b,
                     batch.reshape(N, 1))
